# baseline (speedup 1.0000x reference)
import jax
import jax.numpy as jnp
from jax.experimental import pallas as pl
from jax.experimental.pallas import tpu as pltpu

NUM_ANCHORS = 4
NUM_LABELS = 5
NC = NUM_ANCHORS
NL = NUM_ANCHORS * NUM_LABELS
NR = NUM_ANCHORS * 2
CHUNK = 1024
NBUF = 8


def _fused_heads_kernel(x_hbm, wc_ref, bc_ref, wl_ref, bl_ref, wr_ref, br_ref,
                        conf_hbm, cls_hbm, reg_hbm,
                        xbuf, cbuf, lbuf, rbuf, w_scr,
                        in_sems, out_sems):
    n = x_hbm.shape[0]
    chunks = n // CHUNK

    w_scr[:, :NC] = wc_ref[...]
    w_scr[:, NC:NC + NL] = wl_ref[...]
    w_scr[:, NC + NL:] = wr_ref[...]

    def in_copy(c, slot):
        return pltpu.make_async_copy(
            x_hbm.at[pl.ds(c * CHUNK, CHUNK), :], xbuf.at[slot],
            in_sems.at[slot])

    def out_copies(c, slot):
        return (
            pltpu.make_async_copy(
                cbuf.at[slot], conf_hbm.at[pl.ds(c * CHUNK, CHUNK), :],
                out_sems.at[slot, 0]),
            pltpu.make_async_copy(
                lbuf.at[slot], cls_hbm.at[pl.ds(c * CHUNK, CHUNK), :],
                out_sems.at[slot, 1]),
            pltpu.make_async_copy(
                rbuf.at[slot], reg_hbm.at[pl.ds(c * CHUNK, CHUNK), :],
                out_sems.at[slot, 2]),
        )

    for k in range(min(NBUF, chunks)):
        in_copy(k, k).start()

    def body(i, _):
        slot = jax.lax.rem(i, NBUF)
        in_copy(i, slot).wait()

        @pl.when(i >= NBUF)
        def _():
            for cp in out_copies(i - NBUF, slot):
                cp.wait()

        acc = jnp.dot(xbuf[slot], w_scr[...],
                      preferred_element_type=jnp.float32)
        cbuf[slot] = acc[:, :NC] + bc_ref[...]
        lbuf[slot] = acc[:, NC:NC + NL] + bl_ref[...]
        rbuf[slot] = acc[:, NC + NL:] + br_ref[...]

        for cp in out_copies(i, slot):
            cp.start()

        @pl.when(i + NBUF < chunks)
        def _():
            in_copy(i + NBUF, slot).start()

        return 0

    jax.lax.fori_loop(0, chunks, body, 0)

    for c in range(chunks - min(NBUF, chunks), chunks):
        for cp in out_copies(c, c % NBUF):
            cp.wait()


def kernel(hidden_states, W_conf, b_conf, W_cls, b_cls, W_reg, b_reg):
    bsz, seq_len, hidden = hidden_states.shape
    x = hidden_states.reshape(bsz * seq_len, hidden)
    n = bsz * seq_len

    def const_spec(r, c):
        return pl.BlockSpec((r, c), lambda: (0, 0))

    any_spec = pl.BlockSpec(memory_space=pltpu.MemorySpace.HBM)

    conf, cls_, reg = pl.pallas_call(
        _fused_heads_kernel,
        in_specs=[
            any_spec,
            const_spec(hidden, NC), const_spec(1, NC),
            const_spec(hidden, NL), const_spec(1, NL),
            const_spec(hidden, NR), const_spec(1, NR),
        ],
        out_specs=[any_spec, any_spec, any_spec],
        out_shape=[
            jax.ShapeDtypeStruct((n, NC), jnp.float32),
            jax.ShapeDtypeStruct((n, NL), jnp.float32),
            jax.ShapeDtypeStruct((n, NR), jnp.float32),
        ],
        scratch_shapes=[
            pltpu.VMEM((NBUF, CHUNK, hidden), jnp.float32),
            pltpu.VMEM((NBUF, CHUNK, NC), jnp.float32),
            pltpu.VMEM((NBUF, CHUNK, NL), jnp.float32),
            pltpu.VMEM((NBUF, CHUNK, NR), jnp.float32),
            pltpu.VMEM((hidden, NC + NL + NR), jnp.float32),
            pltpu.SemaphoreType.DMA((NBUF,)),
            pltpu.SemaphoreType.DMA((NBUF, 3)),
        ],
    )(x, W_conf, b_conf.reshape(1, NC), W_cls, b_cls.reshape(1, NL),
      W_reg, b_reg.reshape(1, NR))

    return (
        conf.reshape(bsz, seq_len, NUM_ANCHORS),
        cls_.reshape(bsz, seq_len, NUM_ANCHORS, NUM_LABELS),
        reg.reshape(bsz, seq_len, NUM_ANCHORS, 2),
    )


# speedup vs baseline: 1.0104x; 1.0104x over previous
import jax
import jax.numpy as jnp
from jax.experimental import pallas as pl
from jax.experimental.pallas import tpu as pltpu

NUM_ANCHORS = 4
NUM_LABELS = 5
NC = NUM_ANCHORS
NL = NUM_ANCHORS * NUM_LABELS
NR = NUM_ANCHORS * 2
BLK = 4096


def _fused_heads_kernel(x_ref, wc_ref, bc_ref, wl_ref, bl_ref, wr_ref, br_ref,
                        conf_ref, cls_ref, reg_ref, w_scr):
    @pl.when(pl.program_id(0) == 0)
    def _():
        w_scr[:, :NC] = wc_ref[...]
        w_scr[:, NC:NC + NL] = wl_ref[...]
        w_scr[:, NC + NL:] = wr_ref[...]

    acc = jnp.dot(x_ref[...], w_scr[...], preferred_element_type=jnp.float32)
    conf_ref[...] = acc[:, :NC] + bc_ref[...]
    cls_ref[...] = acc[:, NC:NC + NL] + bl_ref[...]
    reg_ref[...] = acc[:, NC + NL:] + br_ref[...]


def kernel(hidden_states, W_conf, b_conf, W_cls, b_cls, W_reg, b_reg):
    bsz, seq_len, hidden = hidden_states.shape
    x = hidden_states.reshape(bsz * seq_len, hidden)
    n = bsz * seq_len

    def const_spec(r, c):
        return pl.BlockSpec((r, c), lambda i: (0, 0))

    conf, cls_, reg = pl.pallas_call(
        _fused_heads_kernel,
        grid=(n // BLK,),
        in_specs=[
            pl.BlockSpec((BLK, hidden), lambda i: (i, 0)),
            const_spec(hidden, NC), const_spec(1, NC),
            const_spec(hidden, NL), const_spec(1, NL),
            const_spec(hidden, NR), const_spec(1, NR),
        ],
        out_specs=[
            pl.BlockSpec((BLK, NC), lambda i: (i, 0)),
            pl.BlockSpec((BLK, NL), lambda i: (i, 0)),
            pl.BlockSpec((BLK, NR), lambda i: (i, 0)),
        ],
        out_shape=[
            jax.ShapeDtypeStruct((n, NC), jnp.float32),
            jax.ShapeDtypeStruct((n, NL), jnp.float32),
            jax.ShapeDtypeStruct((n, NR), jnp.float32),
        ],
        scratch_shapes=[pltpu.VMEM((hidden, NC + NL + NR), jnp.float32)],
        compiler_params=pltpu.CompilerParams(
            dimension_semantics=("arbitrary",),
            skip_device_barrier=True,
        ),
    )(x, W_conf, b_conf.reshape(1, NC), W_cls, b_cls.reshape(1, NL),
      W_reg, b_reg.reshape(1, NR))

    return (
        conf.reshape(bsz, seq_len, NUM_ANCHORS),
        cls_.reshape(bsz, seq_len, NUM_ANCHORS, NUM_LABELS),
        reg.reshape(bsz, seq_len, NUM_ANCHORS, 2),
    )


# speedup vs baseline: 1.0207x; 1.0102x over previous
import jax
import jax.numpy as jnp
from jax.experimental import pallas as pl
from jax.experimental.pallas import tpu as pltpu

NUM_ANCHORS = 4
NUM_LABELS = 5
NC = NUM_ANCHORS
NL = NUM_ANCHORS * NUM_LABELS
NR = NUM_ANCHORS * 2
BLK = 4096


def _fused_heads_kernel(x_ref, wc_ref, bc_ref, wl_ref, bl_ref, wr_ref, br_ref,
                        conf_ref, cls_ref, reg_ref, w_scr):
    @pl.when(pl.program_id(0) == 0)
    def _():
        w_scr[:, :NC] = wc_ref[...]
        w_scr[:, NC:NC + NL] = wl_ref[...]
        w_scr[:, NC + NL:] = wr_ref[...]

    acc = jnp.dot(x_ref[...], w_scr[...], preferred_element_type=jnp.float32)
    conf_ref[...] = acc[:, :NC] + bc_ref[...]
    cls_ref[...] = acc[:, NC:NC + NL] + bl_ref[...]
    reg_ref[...] = acc[:, NC + NL:] + br_ref[...]


def kernel(hidden_states, W_conf, b_conf, W_cls, b_cls, W_reg, b_reg):
    bsz, seq_len, hidden = hidden_states.shape
    x = hidden_states.reshape(bsz * seq_len, hidden)
    n = bsz * seq_len

    def const_spec(r, c):
        return pl.BlockSpec((r, c), lambda i: (0, 0))

    conf, cls_, reg = pl.pallas_call(
        _fused_heads_kernel,
        grid=(n // BLK,),
        in_specs=[
            pl.BlockSpec((BLK, hidden), lambda i: (i, 0)),
            const_spec(hidden, NC), const_spec(1, NC),
            const_spec(hidden, NL), const_spec(1, NL),
            const_spec(hidden, NR), const_spec(1, NR),
        ],
        out_specs=[
            pl.BlockSpec((BLK, NC), lambda i: (i, 0)),
            pl.BlockSpec((BLK, NL), lambda i: (i, 0)),
            pl.BlockSpec((BLK, NR), lambda i: (i, 0)),
        ],
        out_shape=[
            jax.ShapeDtypeStruct((n, NC), jnp.float32),
            jax.ShapeDtypeStruct((n, NL), jnp.float32),
            jax.ShapeDtypeStruct((n, NR), jnp.float32),
        ],
        scratch_shapes=[pltpu.VMEM((hidden, NC + NL + NR), jnp.float32)],
        compiler_params=pltpu.CompilerParams(
            dimension_semantics=("parallel",),
            skip_device_barrier=True,
        ),
    )(x, W_conf, b_conf.reshape(1, NC), W_cls, b_cls.reshape(1, NL),
      W_reg, b_reg.reshape(1, NR))

    return (
        conf.reshape(bsz, seq_len, NUM_ANCHORS),
        cls_.reshape(bsz, seq_len, NUM_ANCHORS, NUM_LABELS),
        reg.reshape(bsz, seq_len, NUM_ANCHORS, 2),
    )


# speedup vs baseline: 1.0254x; 1.0046x over previous
import jax
import jax.numpy as jnp
from jax.experimental import pallas as pl
from jax.experimental.pallas import tpu as pltpu

NUM_ANCHORS = 4
NUM_LABELS = 5
NC = NUM_ANCHORS
NL = NUM_ANCHORS * NUM_LABELS
NR = NUM_ANCHORS * 2
BLK = 4096


def _fused_heads_kernel(x_ref, wc_ref, bc_ref, wl_ref, bl_ref, wr_ref, br_ref,
                        conf_ref, cls_ref, reg_ref, w_scr):
    @pl.when(pl.program_id(0) == 0)
    def _():
        w_scr[:, :NC] = wc_ref[...]
        w_scr[:, NC:NC + NL] = wl_ref[...]
        w_scr[:, NC + NL:] = wr_ref[...]

    acc = jnp.dot(x_ref[...], w_scr[...], preferred_element_type=jnp.float32)
    conf_ref[...] = acc[:, :NC] + bc_ref[...]
    cls_ref[...] = acc[:, NC:NC + NL] + bl_ref[...]
    reg_ref[...] = acc[:, NC + NL:] + br_ref[...]


def kernel(hidden_states, W_conf, b_conf, W_cls, b_cls, W_reg, b_reg):
    bsz, seq_len, hidden = hidden_states.shape
    x = hidden_states.reshape(bsz * seq_len, hidden)
    n = bsz * seq_len

    def const_spec(r, c):
        return pl.BlockSpec((r, c), lambda i: (0, 0))

    conf, cls_, reg = pl.pallas_call(
        _fused_heads_kernel,
        grid=(n // BLK,),
        in_specs=[
            pl.BlockSpec((BLK, hidden), lambda i: (i, 0)),
            const_spec(hidden, NC), const_spec(1, NC),
            const_spec(hidden, NL), const_spec(1, NL),
            const_spec(hidden, NR), const_spec(1, NR),
        ],
        out_specs=[
            pl.BlockSpec((BLK, NC), lambda i: (i, 0)),
            pl.BlockSpec((BLK, NL), lambda i: (i, 0)),
            pl.BlockSpec((BLK, NR), lambda i: (i, 0)),
        ],
        out_shape=[
            jax.ShapeDtypeStruct((n, NC), jnp.float32),
            jax.ShapeDtypeStruct((n, NL), jnp.float32),
            jax.ShapeDtypeStruct((n, NR), jnp.float32),
        ],
        scratch_shapes=[pltpu.VMEM((hidden, NC + NL + NR), jnp.float32)],
        compiler_params=pltpu.CompilerParams(
            dimension_semantics=("parallel",),
        ),
    )(x, W_conf, b_conf.reshape(1, NC), W_cls, b_cls.reshape(1, NL),
      W_reg, b_reg.reshape(1, NR))

    return (
        conf.reshape(bsz, seq_len, NUM_ANCHORS),
        cls_.reshape(bsz, seq_len, NUM_ANCHORS, NUM_LABELS),
        reg.reshape(bsz, seq_len, NUM_ANCHORS, 2),
    )
